# baseline (speedup 1.0000x reference)
import functools

import jax
import jax.numpy as jnp
from jax import lax
from jax.experimental import pallas as pl
from jax.experimental.pallas import tpu as pltpu, tpu_sc as plsc

B = 16384
DIM = 64
K = 128


@functools.lru_cache(maxsize=None)
def _build(num_cores, num_subcores):
    NW = num_cores * num_subcores
    b_per_w = B // NW
    C = b_per_w // K

    mesh = plsc.VectorSubcoreMesh(core_axis_name="c", subcore_axis_name="s")
    out_sds = jax.ShapeDtypeStruct((B, DIM), jnp.float32)

    @functools.partial(
        pl.kernel,
        mesh=mesh,
        out_type=(out_sds, out_sds, out_sds),
        scratch_types=[
            pltpu.VMEM((C, K), jnp.int32),
            pltpu.VMEM((C, K), jnp.int32),
            pltpu.VMEM((C, K), jnp.int32),
            pltpu.VMEM((b_per_w, DIM), jnp.float32),
            pltpu.VMEM((b_per_w, DIM), jnp.float32),
            pltpu.VMEM((b_per_w, DIM), jnp.float32),
            pltpu.SemaphoreType.DMA,
            pltpu.SemaphoreType.DMA,
        ],
        compiler_params=pltpu.CompilerParams(use_tc_tiling_on_sc=False),
    )
    def sc_gather3(user_i, pos_i, neg_i, utab, itab, out_u, out_p, out_n,
                   uidx, pidx, nidx, urows, prows, nrows, gsem, wsem):
        wid = lax.axis_index("s") * num_cores + lax.axis_index("c")
        base = wid * b_per_w

        pltpu.sync_copy(user_i.at[pl.ds(wid * C, C)], uidx)
        pltpu.sync_copy(pos_i.at[pl.ds(wid * C, C)], pidx)
        pltpu.sync_copy(neg_i.at[pl.ds(wid * C, C)], nidx)

        copies = []
        for tab, idx, rows in ((utab, uidx, urows),
                               (itab, pidx, prows),
                               (itab, nidx, nrows)):
            for c in range(C):
                copies.append(
                    pltpu.async_copy(tab.at[idx.at[c]],
                                     rows.at[pl.ds(c * K, K)], gsem))
        for cp in copies:
            cp.wait()

        wcopies = [
            pltpu.async_copy(urows, out_u.at[pl.ds(base, b_per_w)], wsem),
            pltpu.async_copy(prows, out_p.at[pl.ds(base, b_per_w)], wsem),
            pltpu.async_copy(nrows, out_n.at[pl.ds(base, b_per_w)], wsem),
        ]
        for cp in wcopies:
            cp.wait()

    return sc_gather3, NW * (b_per_w // K), K


def kernel(user, pos, neg, user_table, item_table):
    info = plsc.get_sparse_core_info()
    fn, nrows, k = _build(info.num_cores, info.num_subcores)
    user2 = user.astype(jnp.int32).reshape(nrows, k)
    pos2 = pos.astype(jnp.int32).reshape(nrows, k)
    neg2 = neg.astype(jnp.int32).reshape(nrows, k)
    return fn(user2, pos2, neg2, user_table, item_table)


# speedup vs baseline: 1.5612x; 1.5612x over previous
import functools

import jax
import jax.numpy as jnp
from jax import lax
from jax.experimental import pallas as pl
from jax.experimental.pallas import tpu as pltpu, tpu_sc as plsc

B = 16384
DIM = 64
CH = 128
NBUF = 3


@functools.lru_cache(maxsize=None)
def _build(num_cores, num_subcores):
    NW = num_cores * num_subcores
    b_per_w = B // NW
    NCH = b_per_w // CH
    G = CH // 16

    mesh = plsc.VectorSubcoreMesh(core_axis_name="c", subcore_axis_name="s")
    out_sds = jax.ShapeDtypeStruct((B, DIM), jnp.float32)

    @functools.partial(
        pl.kernel,
        mesh=mesh,
        out_type=(out_sds, out_sds, out_sds),
        scratch_types=[
            pltpu.VMEM((b_per_w,), jnp.int32),
            pltpu.VMEM((b_per_w,), jnp.int32),
            pltpu.VMEM((b_per_w,), jnp.int32),
            [pltpu.VMEM((CH, DIM), jnp.float32) for _ in range(NBUF)],
            [pltpu.SemaphoreType.DMA for _ in range(NBUF)],
            [pltpu.SemaphoreType.DMA for _ in range(NBUF)],
        ],
    )
    def sc_gather3(u_i, p_i, n_i, utab, itab, out_u, out_p, out_n,
                   uidx, pidx, nidx, bufs, gsems, wsems):
        wid = lax.axis_index("s") * num_cores + lax.axis_index("c")
        base = wid * b_per_w

        pltpu.sync_copy(u_i.at[wid], uidx)
        pltpu.sync_copy(p_i.at[wid], pidx)
        pltpu.sync_copy(n_i.at[wid], nidx)

        sched = []
        for tab, idx, out in ((utab, uidx, out_u),
                              (itab, pidx, out_p),
                              (itab, nidx, out_n)):
            for c in range(NCH):
                sched.append((tab, idx, out, c * CH))
        total = len(sched)

        def fire(slot):
            tab, idx, _, ofs = sched[slot]
            buf = bufs[slot % NBUF]
            sem = gsems[slot % NBUF]

            def issue(g, carry):
                v = idx[pl.ds(ofs + g * 16, 16)]
                for l in range(16):
                    i = v[l]
                    pltpu.async_copy(tab.at[pl.ds(i, 1)],
                                     buf.at[pl.ds(g * 16 + l, 1)], sem)
                return carry

            lax.fori_loop(0, G, issue, 0)

        def drain_gathers(slot):
            tab = sched[slot][0]
            buf = bufs[slot % NBUF]
            sem = gsems[slot % NBUF]

            def one(j, carry):
                pltpu.make_async_copy(tab.at[pl.ds(0, 1)],
                                      buf.at[pl.ds(0, 1)], sem).wait()
                return carry

            lax.fori_loop(0, CH, one, 0)

        def start_writeback(slot):
            _, _, out, ofs = sched[slot]
            buf = bufs[slot % NBUF]
            pltpu.async_copy(buf, out.at[pl.ds(base + ofs, CH)], wsems[slot % NBUF])

        def drain_writeback(slot):
            _, _, out, ofs = sched[slot]
            buf = bufs[slot % NBUF]
            pltpu.make_async_copy(buf, out.at[pl.ds(base + ofs, CH)],
                                  wsems[slot % NBUF]).wait()

        for s in range(min(NBUF - 1, total)):
            fire(s)
        for s in range(total):
            drain_gathers(s)
            start_writeback(s)
            nxt = s + NBUF - 1
            if nxt < total:
                prev = nxt - NBUF
                if prev >= 0:
                    drain_writeback(prev)
                fire(nxt)
        for s in range(total - NBUF, total):
            if s >= 0:
                drain_writeback(s)

    return sc_gather3, NW, b_per_w


def kernel(user, pos, neg, user_table, item_table):
    info = plsc.get_sparse_core_info()
    fn, nw, bw = _build(info.num_cores, info.num_subcores)
    u = user.astype(jnp.int32).reshape(nw, bw)
    p = pos.astype(jnp.int32).reshape(nw, bw)
    n = neg.astype(jnp.int32).reshape(nw, bw)
    return fn(u, p, n, user_table, item_table)
